# Initial kernel scaffold; baseline (speedup 1.0000x reference)
import functools

import jax
import jax.numpy as jnp
from jax import lax
from jax.experimental import pallas as pl
from jax.experimental.pallas import tpu as pltpu
from jax.experimental.pallas import tpu_sc as plsc

N = 50000
E = 800000
HID = 64
TD = 32

NH = 25000
A_ROWS = 25008
WU = 72
BE = 3200
EG = E // BE
BN = 2500
NG = N // BN
NHB = NH // BN
GCH = 512
PER_TILE = E // 32
GFULL = PER_TILE // GCH
NB = E // 128
STRIPE = A_ROWS // 16

_F32 = jnp.float32


def _sc_mesh():
  return plsc.VectorSubcoreMesh(core_axis_name="c", subcore_axis_name="s")


def _te_encode(dt, te_w, te_b):
  def body(dt_ref, w_ref, b_ref, o_ref):
    o_ref[...] = jnp.cos(dt_ref[...][:, None] * w_ref[...][None, :]
                         + b_ref[...][None, :])

  return pl.pallas_call(
      body,
      grid=(EG,),
      in_specs=[
          pl.BlockSpec((BE,), lambda i: (i,)),
          pl.BlockSpec((TD,), lambda i: (0,)),
          pl.BlockSpec((TD,), lambda i: (0,)),
      ],
      out_specs=pl.BlockSpec((BE, TD), lambda i: (i, 0)),
      out_shape=jax.ShapeDtypeStruct((E, TD), _F32),
  )(dt, te_w, te_b)


def _precompute(h, Wvh, Wk, WkT, WvtT, Wq):
  din = h.shape[1]

  def body(h_ref, wvh_ref, wk_ref, wkt_ref, wvtt_ref, wq_ref, stab_ref, dtab_ref):
    hb = h_ref[...]
    p = jnp.dot(hb, wvh_ref[...], preferred_element_type=_F32)
    kh = jnp.dot(p, wk_ref[...], preferred_element_type=_F32) * 0.125
    q = jnp.dot(hb, wq_ref[...], preferred_element_type=_F32)
    r = jnp.dot(jnp.dot(q, wkt_ref[...], preferred_element_type=_F32),
                wvtt_ref[...], preferred_element_type=_F32) * 0.125
    stab_ref[:, :HID] = kh
    stab_ref[:, HID:] = p
    dtab_ref[:, :HID] = q
    dtab_ref[:, HID:] = r

  return pl.pallas_call(
      body,
      grid=(NG,),
      in_specs=[
          pl.BlockSpec((BN, din), lambda i: (i, 0)),
          pl.BlockSpec((din, HID), lambda i: (0, 0)),
          pl.BlockSpec((HID, HID), lambda i: (0, 0)),
          pl.BlockSpec((HID, HID), lambda i: (0, 0)),
          pl.BlockSpec((HID, TD), lambda i: (0, 0)),
          pl.BlockSpec((din, HID), lambda i: (0, 0)),
      ],
      out_specs=[
          pl.BlockSpec((BN, 2 * HID), lambda i: (i, 0)),
          pl.BlockSpec((BN, HID + TD), lambda i: (i, 0)),
      ],
      out_shape=[
          jax.ShapeDtypeStruct((N, 2 * HID), _F32),
          jax.ShapeDtypeStruct((N, HID + TD), _F32),
      ],
  )(h, Wvh, Wk, WkT, WvtT, Wq)


def _gather(src_tab, dst_tab, src_idx, dst_idx):
  @functools.partial(
      pl.kernel,
      out_type=(
          jax.ShapeDtypeStruct((E, 2 * HID), _F32),
          jax.ShapeDtypeStruct((E, HID + TD), _F32),
      ),
      mesh=_sc_mesh(),
      scratch_types=[
          pltpu.VMEM((GCH,), jnp.int32),
          pltpu.VMEM((GCH, 2 * HID), _F32),
          pltpu.VMEM((GCH, HID + TD), _F32),
      ],
  )
  def k(stab_h, dtab_h, sidx_h, didx_h, gsrc_h, gdst_h, idx_v, srow_v, drow_v):
    cc = lax.axis_index("c")
    ss = lax.axis_index("s")
    base0 = (ss * 2 + cc) * PER_TILE

    @pl.loop(0, GFULL + 1)
    def _(it):
      b = base0 + jnp.minimum(it * GCH, PER_TILE - GCH)
      pltpu.sync_copy(sidx_h.at[pl.ds(b, GCH)], idx_v)
      pltpu.sync_copy(stab_h.at[idx_v], srow_v)
      pltpu.sync_copy(srow_v, gsrc_h.at[pl.ds(b, GCH)])
      pltpu.sync_copy(didx_h.at[pl.ds(b, GCH)], idx_v)
      pltpu.sync_copy(dtab_h.at[idx_v], drow_v)
      pltpu.sync_copy(drow_v, gdst_h.at[pl.ds(b, GCH)])

  return k(src_tab, dst_tab, src_idx, dst_idx)


def _logits(g_src, g_dst, te):
  def body(gs_ref, gd_ref, te_ref, l_ref, mg_ref):
    i = pl.program_id(0)
    gd = gd_ref[...]
    l = (jnp.sum(gs_ref[...] * gd[:, :HID], axis=1)
         + jnp.sum(gd[:, HID:] * te_ref[...], axis=1))
    l_ref[...] = l
    bm = jnp.max(l)

    @pl.when(i == 0)
    def _():
      mg_ref[0, 0] = bm

    @pl.when(i > 0)
    def _():
      mg_ref[0, 0] = jnp.maximum(mg_ref[0, 0], bm)

  return pl.pallas_call(
      body,
      grid=(EG,),
      in_specs=[
          pl.BlockSpec((BE, HID), lambda i: (i, 0)),
          pl.BlockSpec((BE, HID + TD), lambda i: (i, 0)),
          pl.BlockSpec((BE, TD), lambda i: (i, 0)),
      ],
      out_specs=[
          pl.BlockSpec((BE,), lambda i: (i,)),
          pl.BlockSpec(memory_space=pltpu.SMEM),
      ],
      out_shape=[
          jax.ShapeDtypeStruct((E,), _F32),
          jax.ShapeDtypeStruct((1, 1), _F32),
      ],
  )(g_src, g_dst, te)


def _updates(logit, mg, g_src, te, Wvt):
  def body(l_ref, mg_ref, gp_ref, te_ref, wvt_ref, u_ref):
    ex = jnp.exp(l_ref[...] - mg_ref[0, 0])
    t = jnp.dot(te_ref[...], wvt_ref[...], preferred_element_type=_F32)
    u_ref[:, :HID] = ex[:, None] * (gp_ref[...] + t)
    u_ref[:, HID:HID + 1] = ex[:, None]
    u_ref[:, HID + 1:] = jnp.zeros((BE, WU - HID - 1), _F32)

  return pl.pallas_call(
      body,
      grid=(EG,),
      in_specs=[
          pl.BlockSpec((BE,), lambda i: (i,)),
          pl.BlockSpec(memory_space=pltpu.SMEM),
          pl.BlockSpec((BE, HID), lambda i: (i, 1)),
          pl.BlockSpec((BE, TD), lambda i: (i, 0)),
          pl.BlockSpec((TD, HID), lambda i: (0, 0)),
      ],
      out_specs=pl.BlockSpec((BE, WU), lambda i: (i, 0)),
      out_shape=jax.ShapeDtypeStruct((E, WU), _F32),
  )(logit, mg, g_src, te, Wvt)


def _scatter(upd, dst_idx, zstripe):
  @functools.partial(
      pl.kernel,
      out_type=jax.ShapeDtypeStruct((2, A_ROWS, WU), _F32),
      mesh=_sc_mesh(),
      scratch_types=[
          pltpu.VMEM_SHARED((A_ROWS, WU), _F32),
          pltpu.VMEM((128,), jnp.int32),
          pltpu.VMEM((1, 128), jnp.int32),
          pltpu.VMEM((128, WU), _F32),
      ],
  )
  def k(upd_h, didx_h, z_h, a_out, a_sh, didx_v, lidx_v, stage_v):
    cc = lax.axis_index("c")
    ss = lax.axis_index("s")
    pltpu.sync_copy(z_h, a_sh.at[pl.ds(ss * STRIPE, STRIPE)])
    plsc.subcore_barrier()
    nbase = cc * NH

    @pl.loop(ss, NB, step=16)
    def _(b):
      e0 = b * 128
      pltpu.sync_copy(didx_h.at[pl.ds(e0, 128)], didx_v)
      for j in range(8):
        d = didx_v[pl.ds(j * 16, 16)]
        rel = d - nbase
        ok = (rel >= 0) & (rel < NH)
        trash = NH + (lax.iota(jnp.int32, 16) & 7)
        lidx_v[0, pl.ds(j * 16, 16)] = jnp.where(ok, rel, trash)
      pltpu.sync_copy(upd_h.at[pl.ds(e0, 128)], stage_v)
      pltpu.sync_copy(stage_v, a_sh.at[lidx_v.at[0]], add=True)

    plsc.subcore_barrier()
    pltpu.sync_copy(a_sh.at[pl.ds(ss * STRIPE, STRIPE)],
                    a_out.at[cc, pl.ds(ss * STRIPE, STRIPE)])

  return k(upd, dst_idx, zstripe)


def _combine(a_out, h, Wo, bo, Wself, bself):
  din = h.shape[1]

  def body(a_ref, h_ref, wo_ref, bo_ref, ws_ref, bs_ref, o_ref):
    a = a_ref[0]
    den = a[:, HID]
    ok = den > 0.0
    dens = jnp.where(ok, den, 1.0)
    agg = jnp.where(ok[:, None], a[:, :HID] / dens[:, None], 0.0)
    o_ref[...] = jax.nn.relu(
        jnp.dot(agg, wo_ref[...], preferred_element_type=_F32)
        + jnp.dot(h_ref[...], ws_ref[...], preferred_element_type=_F32)
        + bo_ref[...] + bs_ref[...])

  return pl.pallas_call(
      body,
      grid=(NG,),
      in_specs=[
          pl.BlockSpec((1, BN, WU), lambda i: (i // NHB, i % NHB, 0)),
          pl.BlockSpec((BN, din), lambda i: (i, 0)),
          pl.BlockSpec((HID, HID), lambda i: (0, 0)),
          pl.BlockSpec((HID,), lambda i: (0,)),
          pl.BlockSpec((din, HID), lambda i: (0, 0)),
          pl.BlockSpec((HID,), lambda i: (0,)),
      ],
      out_specs=pl.BlockSpec((BN, HID), lambda i: (i, 0)),
      out_shape=jax.ShapeDtypeStruct((N, HID), _F32),
  )(a_out, h, Wo, bo, Wself, bself)


def _readout(h, S1, sb1, S2, sb2):
  def body(h_ref, s1_ref, sb1_ref, s2_ref, sb2_ref, o_ref, acc_ref):
    i = pl.program_id(0)

    @pl.when(i == 0)
    def _():
      acc_ref[...] = jnp.zeros((1, HID), _F32)

    acc_ref[...] += jnp.sum(h_ref[...], axis=0, keepdims=True)

    @pl.when(i == NG - 1)
    def _():
      hg = acc_ref[...] * (1.0 / N)
      z = jax.nn.relu(jnp.dot(hg, s1_ref[...], preferred_element_type=_F32)
                      + sb1_ref[...])
      o_ref[...] = (jnp.dot(z, s2_ref[...], preferred_element_type=_F32)
                    + sb2_ref[...])

  return pl.pallas_call(
      body,
      grid=(NG,),
      in_specs=[
          pl.BlockSpec((BN, HID), lambda i: (i, 0)),
          pl.BlockSpec((HID, HID), lambda i: (0, 0)),
          pl.BlockSpec((HID,), lambda i: (0,)),
          pl.BlockSpec((HID, 1), lambda i: (0, 0)),
          pl.BlockSpec((1, 1), lambda i: (0, 0)),
      ],
      out_specs=pl.BlockSpec((1, 1), lambda i: (0, 0)),
      out_shape=jax.ShapeDtypeStruct((1, 1), _F32),
      scratch_shapes=[pltpu.VMEM((1, HID), _F32)],
  )(h, S1, sb1, S2, sb2)


def kernel(edge_index, dt, u_mask, v_mask, te_w, te_b,
           Wv0, Wk0, Wq0, Wo0, bo0, Wself0, bself0,
           Wv1, Wk1, Wq1, Wo1, bo1, Wself1, bself1,
           S1, sb1, S2, sb2):
  src = edge_index[0]
  dst = edge_index[1]
  feat = jnp.stack([u_mask.astype(_F32), v_mask.astype(_F32)], axis=-1)
  te = _te_encode(dt, te_w, te_b)
  zstripe = jnp.zeros((STRIPE, WU), _F32)

  h = feat
  for Wv, Wk, Wq, Wo, bo, Wself, bself in (
      (Wv0, Wk0, Wq0, Wo0, bo0, Wself0, bself0),
      (Wv1, Wk1, Wq1, Wo1, bo1, Wself1, bself1),
  ):
    din = Wq.shape[0]
    Wvh = Wv[:din]
    Wvt = Wv[din:]
    src_tab, dst_tab = _precompute(h, Wvh, Wk, Wk.T, Wvt.T, Wq)
    g_src, g_dst = _gather(src_tab, dst_tab, src, dst)
    logit, mg = _logits(g_src, g_dst, te)
    upd = _updates(logit, mg, g_src, te, Wvt)
    a_out = _scatter(upd, dst, zstripe)
    h = _combine(a_out, h, Wo, bo, Wself, bself)

  out = _readout(h, S1, sb1, S2, sb2.reshape(1, 1))
  return out.reshape(1)


# speedup vs baseline: 4.1944x; 4.1944x over previous
import functools

import jax
import jax.numpy as jnp
from jax import lax
from jax.experimental import pallas as pl
from jax.experimental.pallas import tpu as pltpu
from jax.experimental.pallas import tpu_sc as plsc

N = 50000
E = 800000
HID = 64
TD = 32

NH = 25000
A_ROWS = 25008
WU = 72
E_PAD = 800768
BE = 2048
EG = E_PAD // BE
BN = 5000
NG = N // BN
NHB = NH // BN
GCH = 512
PER_TILE = E_PAD // 32
GFULL = PER_TILE // GCH
NB = E // 128
STRIPE = A_ROWS // 16

_F32 = jnp.float32


def _sc_mesh():
  return plsc.VectorSubcoreMesh(core_axis_name="c", subcore_axis_name="s")


def _te_encode(dt2, te_w2, te_b2):
  def body(dt_ref, w_ref, b_ref, o_ref):
    o_ref[...] = jnp.cos(dt_ref[...] * w_ref[...] + b_ref[...])

  return pl.pallas_call(
      body,
      grid=(EG,),
      in_specs=[
          pl.BlockSpec((BE, 1), lambda i: (i, 0)),
          pl.BlockSpec((1, TD), lambda i: (0, 0)),
          pl.BlockSpec((1, TD), lambda i: (0, 0)),
      ],
      out_specs=pl.BlockSpec((BE, TD), lambda i: (i, 0)),
      out_shape=jax.ShapeDtypeStruct((E_PAD, TD), _F32),
  )(dt2, te_w2, te_b2)


def _precompute(h, Wvh, Wk, WkT, WvtT, Wq):
  din = h.shape[1]

  def body(h_ref, wvh_ref, wk_ref, wkt_ref, wvtt_ref, wq_ref, stab_ref, dtab_ref):
    hb = h_ref[...]
    p = jnp.dot(hb, wvh_ref[...], preferred_element_type=_F32)
    kh = jnp.dot(p, wk_ref[...], preferred_element_type=_F32) * 0.125
    q = jnp.dot(hb, wq_ref[...], preferred_element_type=_F32)
    r = jnp.dot(jnp.dot(q, wkt_ref[...], preferred_element_type=_F32),
                wvtt_ref[...], preferred_element_type=_F32) * 0.125
    stab_ref[:, :HID] = kh
    stab_ref[:, HID:] = p
    dtab_ref[:, :HID] = q
    dtab_ref[:, HID:] = r

  return pl.pallas_call(
      body,
      grid=(NG,),
      in_specs=[
          pl.BlockSpec((BN, din), lambda i: (i, 0)),
          pl.BlockSpec((din, HID), lambda i: (0, 0)),
          pl.BlockSpec((HID, HID), lambda i: (0, 0)),
          pl.BlockSpec((HID, HID), lambda i: (0, 0)),
          pl.BlockSpec((HID, TD), lambda i: (0, 0)),
          pl.BlockSpec((din, HID), lambda i: (0, 0)),
      ],
      out_specs=[
          pl.BlockSpec((BN, 2 * HID), lambda i: (i, 0)),
          pl.BlockSpec((BN, HID + TD), lambda i: (i, 0)),
      ],
      out_shape=[
          jax.ShapeDtypeStruct((N, 2 * HID), _F32),
          jax.ShapeDtypeStruct((N, HID + TD), _F32),
      ],
  )(h, Wvh, Wk, WkT, WvtT, Wq)


def _gather(src_tab, dst_tab, src_idx, dst_idx):
  @functools.partial(
      pl.kernel,
      out_type=(
          jax.ShapeDtypeStruct((E_PAD, 2 * HID), _F32),
          jax.ShapeDtypeStruct((E_PAD, HID + TD), _F32),
      ),
      mesh=_sc_mesh(),
      compiler_params=pltpu.CompilerParams(use_tc_tiling_on_sc=False),
      scratch_types=[
          pltpu.VMEM((GCH,), jnp.int32),
          pltpu.VMEM((GCH, 2 * HID), _F32),
          pltpu.VMEM((GCH, HID + TD), _F32),
      ],
  )
  def k(stab_h, dtab_h, sidx_h, didx_h, gsrc_h, gdst_h, idx_v, srow_v, drow_v):
    cc = lax.axis_index("c")
    ss = lax.axis_index("s")
    base0 = (ss * 2 + cc) * PER_TILE

    @pl.loop(0, GFULL + 1)
    def _(it):
      b = base0 + jnp.minimum(it * GCH, PER_TILE - GCH)
      pltpu.sync_copy(sidx_h.at[pl.ds(b, GCH)], idx_v)
      pltpu.sync_copy(stab_h.at[idx_v], srow_v)
      pltpu.sync_copy(srow_v, gsrc_h.at[pl.ds(b, GCH)])
      pltpu.sync_copy(didx_h.at[pl.ds(b, GCH)], idx_v)
      pltpu.sync_copy(dtab_h.at[idx_v], drow_v)
      pltpu.sync_copy(drow_v, gdst_h.at[pl.ds(b, GCH)])

  return k(src_tab, dst_tab, src_idx, dst_idx)


def _logits(g_src, g_dst, te):
  def body(gs_ref, gd_ref, te_ref, l_ref, mg_ref):
    i = pl.program_id(0)
    gd = gd_ref[...]
    l = (jnp.sum(gs_ref[...][:, :HID] * gd[:, :HID], axis=1, keepdims=True)
         + jnp.sum(gd[:, HID:] * te_ref[...], axis=1, keepdims=True))
    l_ref[...] = l
    bm = jnp.max(l)

    @pl.when(i == 0)
    def _():
      mg_ref[0, 0] = bm

    @pl.when(i > 0)
    def _():
      mg_ref[0, 0] = jnp.maximum(mg_ref[0, 0], bm)

  return pl.pallas_call(
      body,
      grid=(EG,),
      in_specs=[
          pl.BlockSpec((BE, 2 * HID), lambda i: (i, 0)),
          pl.BlockSpec((BE, HID + TD), lambda i: (i, 0)),
          pl.BlockSpec((BE, TD), lambda i: (i, 0)),
      ],
      out_specs=[
          pl.BlockSpec((BE, 1), lambda i: (i, 0)),
          pl.BlockSpec(memory_space=pltpu.SMEM),
      ],
      out_shape=[
          jax.ShapeDtypeStruct((E_PAD, 1), _F32),
          jax.ShapeDtypeStruct((1, 1), _F32),
      ],
  )(g_src, g_dst, te)


def _updates(logit, mg, g_src, te, Wvt):
  def body(l_ref, mg_ref, gs_ref, te_ref, wvt_ref, u_ref):
    ex = jnp.exp(l_ref[...] - mg_ref[0, 0])
    t = jnp.dot(te_ref[...], wvt_ref[...], preferred_element_type=_F32)
    u_ref[:, :HID] = ex * (gs_ref[...][:, HID:] + t)
    u_ref[:, HID:HID + 1] = ex
    u_ref[:, HID + 1:] = jnp.zeros((BE, WU - HID - 1), _F32)

  return pl.pallas_call(
      body,
      grid=(EG,),
      in_specs=[
          pl.BlockSpec((BE, 1), lambda i: (i, 0)),
          pl.BlockSpec(memory_space=pltpu.SMEM),
          pl.BlockSpec((BE, 2 * HID), lambda i: (i, 0)),
          pl.BlockSpec((BE, TD), lambda i: (i, 0)),
          pl.BlockSpec((TD, HID), lambda i: (0, 0)),
      ],
      out_specs=pl.BlockSpec((BE, WU), lambda i: (i, 0)),
      out_shape=jax.ShapeDtypeStruct((E_PAD, WU), _F32),
  )(logit, mg, g_src, te, Wvt)


def _scatter(upd, dst_idx, zstripe):
  @functools.partial(
      pl.kernel,
      out_type=jax.ShapeDtypeStruct((2, A_ROWS, WU), _F32),
      mesh=_sc_mesh(),
      compiler_params=pltpu.CompilerParams(use_tc_tiling_on_sc=False),
      scratch_types=[
          pltpu.VMEM_SHARED((A_ROWS, WU), _F32),
          pltpu.VMEM((128,), jnp.int32),
          pltpu.VMEM((1, 128), jnp.int32),
          pltpu.VMEM((128, WU), _F32),
      ],
  )
  def k(upd_h, didx_h, z_h, a_out, a_sh, didx_v, lidx_v, stage_v):
    cc = lax.axis_index("c")
    ss = lax.axis_index("s")
    pltpu.sync_copy(z_h, a_sh.at[pl.ds(ss * STRIPE, STRIPE)])
    plsc.subcore_barrier()
    nbase = cc * NH

    @pl.loop(ss, NB, step=16)
    def _(b):
      e0 = b * 128
      pltpu.sync_copy(didx_h.at[pl.ds(e0, 128)], didx_v)
      for j in range(8):
        d = didx_v[pl.ds(j * 16, 16)]
        rel = d - nbase
        ok = (rel >= 0) & (rel < NH)
        trash = NH + (lax.iota(jnp.int32, 16) & 7)
        lidx_v[0, pl.ds(j * 16, 16)] = jnp.where(ok, rel, trash)
      pltpu.sync_copy(upd_h.at[pl.ds(e0, 128)], stage_v)
      pltpu.sync_copy(stage_v, a_sh.at[lidx_v.at[0]], add=True)

    plsc.subcore_barrier()
    pltpu.sync_copy(a_sh.at[pl.ds(ss * STRIPE, STRIPE)],
                    a_out.at[cc, pl.ds(ss * STRIPE, STRIPE)])

  return k(upd, dst_idx, zstripe)


def _combine(a_out, h, Wo, bo, Wself, bself):
  din = h.shape[1]

  def body(a_ref, h_ref, wo_ref, bo_ref, ws_ref, bs_ref, o_ref):
    a = a_ref[0]
    den = a[:, HID]
    ok = den > 0.0
    dens = jnp.where(ok, den, 1.0)
    agg = jnp.where(ok[:, None], a[:, :HID] / dens[:, None], 0.0)
    o_ref[...] = jax.nn.relu(
        jnp.dot(agg, wo_ref[...], preferred_element_type=_F32)
        + jnp.dot(h_ref[...], ws_ref[...], preferred_element_type=_F32)
        + bo_ref[...] + bs_ref[...])

  return pl.pallas_call(
      body,
      grid=(NG,),
      in_specs=[
          pl.BlockSpec((1, BN, WU), lambda i: (i // NHB, i % NHB, 0)),
          pl.BlockSpec((BN, din), lambda i: (i, 0)),
          pl.BlockSpec((HID, HID), lambda i: (0, 0)),
          pl.BlockSpec((HID,), lambda i: (0,)),
          pl.BlockSpec((din, HID), lambda i: (0, 0)),
          pl.BlockSpec((HID,), lambda i: (0,)),
      ],
      out_specs=pl.BlockSpec((BN, HID), lambda i: (i, 0)),
      out_shape=jax.ShapeDtypeStruct((N, HID), _F32),
  )(a_out, h, Wo, bo, Wself, bself)


def _readout(h, S1, sb1, S2, sb2):
  def body(h_ref, s1_ref, sb1_ref, s2_ref, sb2_ref, o_ref, acc_ref):
    i = pl.program_id(0)

    @pl.when(i == 0)
    def _():
      acc_ref[...] = jnp.zeros((1, HID), _F32)

    acc_ref[...] += jnp.sum(h_ref[...], axis=0, keepdims=True)

    @pl.when(i == NG - 1)
    def _():
      hg = acc_ref[...] * (1.0 / N)
      z = jax.nn.relu(jnp.dot(hg, s1_ref[...], preferred_element_type=_F32)
                      + sb1_ref[...])
      o_ref[...] = (jnp.dot(z, s2_ref[...], preferred_element_type=_F32)
                    + sb2_ref[...])

  return pl.pallas_call(
      body,
      grid=(NG,),
      in_specs=[
          pl.BlockSpec((BN, HID), lambda i: (i, 0)),
          pl.BlockSpec((HID, HID), lambda i: (0, 0)),
          pl.BlockSpec((HID,), lambda i: (0,)),
          pl.BlockSpec((HID, 1), lambda i: (0, 0)),
          pl.BlockSpec((1, 1), lambda i: (0, 0)),
      ],
      out_specs=pl.BlockSpec((1, 1), lambda i: (0, 0)),
      out_shape=jax.ShapeDtypeStruct((1, 1), _F32),
      scratch_shapes=[pltpu.VMEM((1, HID), _F32)],
  )(h, S1, sb1, S2, sb2)


def kernel(edge_index, dt, u_mask, v_mask, te_w, te_b,
           Wv0, Wk0, Wq0, Wo0, bo0, Wself0, bself0,
           Wv1, Wk1, Wq1, Wo1, bo1, Wself1, bself1,
           S1, sb1, S2, sb2):
  src = edge_index[0]
  dst = edge_index[1]
  pad = E_PAD - E
  src_p = jnp.pad(src, (0, pad))
  dst_p = jnp.pad(dst, (0, pad))
  dt_p = jnp.pad(dt, (0, pad))
  feat = jnp.stack([u_mask.astype(_F32), v_mask.astype(_F32)], axis=-1)
  te = _te_encode(dt_p.reshape(E_PAD, 1), te_w.reshape(1, TD),
                  te_b.reshape(1, TD))
  zstripe = jnp.zeros((STRIPE, WU), _F32)

  h = feat
  for Wv, Wk, Wq, Wo, bo, Wself, bself in (
      (Wv0, Wk0, Wq0, Wo0, bo0, Wself0, bself0),
      (Wv1, Wk1, Wq1, Wo1, bo1, Wself1, bself1),
  ):
    din = Wq.shape[0]
    Wvh = Wv[:din]
    Wvt = Wv[din:]
    src_tab, dst_tab = _precompute(h, Wvh, Wk, Wk.T, Wvt.T, Wq)
    g_src, g_dst = _gather(src_tab, dst_tab, src_p, dst_p)
    logit, mg = _logits(g_src, g_dst, te)
    upd = _updates(logit, mg, g_src, te, Wvt)
    a_out = _scatter(upd, dst, zstripe)
    h = _combine(a_out, h, Wo, bo, Wself, bself)

  out = _readout(h, S1, sb1, S2, sb2.reshape(1, 1))
  return out.reshape(1)
